# baseline (speedup 1.0000x reference)
import functools

import jax
import jax.numpy as jnp
from jax import lax
from jax.experimental import pallas as pl
from jax.experimental.pallas import tpu as pltpu
from jax.experimental.pallas import tpu_sc as plsc


def _make_copy_kernel(rows: int, cols: int):
    n_workers = 32
    rows_per_w = rows // n_workers
    nbuf = 8
    chunk = rows_per_w // nbuf

    mesh = plsc.VectorSubcoreMesh(core_axis_name="c", subcore_axis_name="s")

    @functools.partial(
        pl.kernel,
        mesh=mesh,
        out_type=jax.ShapeDtypeStruct((rows, cols), jnp.float32),
        scratch_types=[
            pltpu.VMEM((nbuf, chunk, cols), jnp.float32),
            pltpu.SemaphoreType.DMA,
            pltpu.SemaphoreType.DMA,
        ],
    )
    def copy_kernel(table_hbm, out_hbm, buf, in_sem, out_sem):
        nc = lax.axis_size("c")
        wid = lax.axis_index("s") * nc + lax.axis_index("c")
        base = wid * rows_per_w
        copies_in = []
        copies_out = []
        for b in range(nbuf):
            copies_in.append(
                pltpu.async_copy(
                    table_hbm.at[pl.ds(base + b * chunk, chunk)],
                    buf.at[b],
                    in_sem,
                )
            )
        for b in range(nbuf):
            copies_in[b].wait()
            copies_out.append(
                pltpu.async_copy(
                    buf.at[b],
                    out_hbm.at[pl.ds(base + b * chunk, chunk)],
                    out_sem,
                )
            )
        for b in range(nbuf):
            copies_out[b].wait()

    return copy_kernel


def kernel(x, pos_table):
    seq_len = x.shape[-1]
    rows, cols = pos_table.shape
    assert seq_len == rows, "positions cover exactly the whole table"
    return _make_copy_kernel(rows, cols)(pos_table)


# speedup vs baseline: 1.0405x; 1.0405x over previous
import functools

import jax
import jax.numpy as jnp
from jax import lax
from jax.experimental import pallas as pl
from jax.experimental.pallas import tpu as pltpu
from jax.experimental.pallas import tpu_sc as plsc


def _make_copy_kernel(rows: int, cols: int):
    n_cores = 2
    rows_per_c = rows // n_cores
    nbuf = 32
    chunk = rows_per_c // nbuf

    mesh = plsc.ScalarSubcoreMesh(axis_name="c", num_cores=n_cores)

    @functools.partial(
        pl.kernel,
        mesh=mesh,
        out_type=jax.ShapeDtypeStruct((rows, cols), jnp.float32),
        scratch_types=[
            pltpu.VMEM_SHARED((nbuf, chunk, cols), jnp.float32),
            pltpu.SemaphoreType.DMA,
            pltpu.SemaphoreType.DMA,
        ],
    )
    def copy_kernel(table_hbm, out_hbm, buf, in_sem, out_sem):
        cid = lax.axis_index("c")
        base = cid * rows_per_c
        copies_in = []
        copies_out = []
        for b in range(nbuf):
            copies_in.append(
                pltpu.async_copy(
                    table_hbm.at[pl.ds(base + b * chunk, chunk)],
                    buf.at[b],
                    in_sem,
                )
            )
        for b in range(nbuf):
            copies_in[b].wait()
            copies_out.append(
                pltpu.async_copy(
                    buf.at[b],
                    out_hbm.at[pl.ds(base + b * chunk, chunk)],
                    out_sem,
                )
            )
        for b in range(nbuf):
            copies_out[b].wait()

    return copy_kernel


def kernel(x, pos_table):
    seq_len = x.shape[-1]
    rows, cols = pos_table.shape
    assert seq_len == rows, "positions cover exactly the whole table"
    return _make_copy_kernel(rows, cols)(pos_table)
